# baseline (speedup 1.0000x reference)
import functools

import jax
import jax.numpy as jnp
from jax import lax
from jax.experimental import pallas as pl
from jax.experimental.pallas import tpu as pltpu
from jax.experimental.pallas import tpu_sc as plsc

N = 10000
E = 320000
G = 16
NC, NS, L = 2, 16, 16
NW = NC * NS
C = 128
CPT = 80
SEG_CPT = 160
N_CHUNKS = NS * SEG_CPT
E_PAD = C * N_CHUNKS
IDX_ROWS = N_CHUNKS
ROWS_PT = 640
N_ACC = ROWS_PT * NS
BLK = 1000
GRID = N // BLK


def _sc_mesh():
    return plsc.VectorSubcoreMesh(
        core_axis_name="c", subcore_axis_name="s",
        num_cores=NC, num_subcores=NS)


def _fill(ref, rows, cols, val):
    def row_body(r, _):
        def col_body(j, __):
            ref[r, pl.ds(j * L, L)] = jnp.full((L,), val, jnp.float32)
            return 0
        return lax.fori_loop(0, cols // L, col_body, 0)
    lax.fori_loop(0, rows, row_body, 0)


NB = 4
LA = 3


def _make_seg_sum(D):
    def body(y_hbm, src_hbm, dst_hbm, out_hbm, src_v, dst_v, acc, *rest):
        rows = rest[:NB]
        sem_g = rest[NB:2 * NB]
        sem_s = rest[2 * NB:]
        cid = lax.axis_index("c")
        sid = lax.axis_index("s")
        cpt = SEG_CPT

        @pl.when(cid == 0)
        def _core0():
            base = sid * cpt
            pltpu.sync_copy(src_hbm.at[pl.ds(base, cpt)], src_v)
            pltpu.sync_copy(dst_hbm.at[pl.ds(base, cpt)], dst_v)
            _fill(rows[0], C, D, 0.0)
            def zero_stripe(j, _):
                pltpu.sync_copy(
                    rows[0], acc.at[pl.ds(sid * ROWS_PT + j * C, C)])
                return 0
            lax.fori_loop(0, ROWS_PT // C, zero_stripe, 0)
            plsc.subcore_barrier()
            for c0 in range(LA):
                pltpu.async_copy(y_hbm.at[src_v.at[c0]], rows[c0], sem_g[c0])
            def round_body(t, _):
                for b in range(NB):
                    ci = t * NB + b
                    bg = (b + LA) % NB
                    cg = ci + LA
                    @pl.when(jnp.logical_and(cg < cpt, cg >= NB))
                    def _():
                        pltpu.make_async_copy(
                            rows[bg], acc.at[dst_v.at[cg - NB]],
                            sem_s[bg]).wait()
                    @pl.when(cg < cpt)
                    def _():
                        pltpu.async_copy(
                            y_hbm.at[src_v.at[cg]], rows[bg], sem_g[bg])
                    pltpu.make_async_copy(
                        y_hbm.at[src_v.at[ci]], rows[b], sem_g[b]).wait()
                    pltpu.async_copy(
                        rows[b], acc.at[dst_v.at[ci]], sem_s[b], add=True)
                return 0
            lax.fori_loop(0, cpt // NB, round_body, 0)
            for b in range(NB):
                pltpu.make_async_copy(
                    rows[b], acc.at[dst_v.at[cpt - NB + b]], sem_s[b]).wait()
            plsc.subcore_barrier()
            def copy_out(j, _):
                r0 = sid * ROWS_PT + j * C
                pltpu.sync_copy(acc.at[pl.ds(r0, C)], rows[0])
                pltpu.sync_copy(rows[0], out_hbm.at[pl.ds(r0, C)])
                return 0
            lax.fori_loop(0, ROWS_PT // C, copy_out, 0)

    return pl.kernel(
        body,
        out_type=jax.ShapeDtypeStruct((N_ACC, D), jnp.float32),
        mesh=_sc_mesh(),
        compiler_params=pltpu.CompilerParams(use_tc_tiling_on_sc=False),
        scratch_types=[
            pltpu.VMEM((SEG_CPT, C), jnp.int32),
            pltpu.VMEM((SEG_CPT, C), jnp.int32),
            pltpu.VMEM_SHARED((N_ACC, D), jnp.float32),
        ] + [pltpu.VMEM((C, D), jnp.float32)] * NB
          + [pltpu.SemaphoreType.DMA] * (2 * NB),
    )


def _make_deg():
    def body(dst_hbm, out_hbm, dst_v, rows_v, acc):
        cid = lax.axis_index("c")
        sid = lax.axis_index("s")
        wid = cid * NS + sid
        pltpu.sync_copy(dst_hbm.at[pl.ds(wid * CPT, CPT)], dst_v)
        _fill(rows_v, C, L, 0.0)
        def zero_stripe(j, _):
            pltpu.sync_copy(rows_v, acc.at[pl.ds(sid * ROWS_PT + j * C, C)])
            return 0
        lax.fori_loop(0, ROWS_PT // C, zero_stripe, 0)
        _fill(rows_v, C, L, 1.0)
        plsc.subcore_barrier()
        def chunk(ci, _):
            pltpu.sync_copy(rows_v, acc.at[dst_v.at[ci]], add=True)
            return 0
        lax.fori_loop(0, CPT, chunk, 0)
        plsc.subcore_barrier()
        def copy_out(j, _):
            r0 = sid * ROWS_PT + j * C
            pltpu.sync_copy(acc.at[pl.ds(r0, C)], rows_v)
            pltpu.sync_copy(rows_v, out_hbm.at[cid, pl.ds(r0, C)])
            return 0
        lax.fori_loop(0, ROWS_PT // C, copy_out, 0)

    return pl.kernel(
        body,
        out_type=jax.ShapeDtypeStruct((NC, N_ACC, L), jnp.float32),
        mesh=_sc_mesh(),
        compiler_params=pltpu.CompilerParams(use_tc_tiling_on_sc=False),
        scratch_types=[
            pltpu.VMEM((CPT, C), jnp.int32),
            pltpu.VMEM((C, L), jnp.float32),
            pltpu.VMEM_SHARED((N_ACC, L), jnp.float32),
        ],
    )


def _tc_pre(x_ref, w1_ref, deg_ref, y1_ref, dis_ref):
    deg = deg_ref[0, :, 0:1] + deg_ref[1, :, 0:1] + 1.0
    dis = lax.rsqrt(deg)
    xw = jnp.dot(x_ref[...], w1_ref[...], preferred_element_type=jnp.float32)
    y1_ref[...] = xw * dis
    dis_ref[...] = dis


def _tc_mid(s1_ref, y1_ref, dis_ref, b1_ref, z_ref):
    s = s1_ref[...]
    dis = dis_ref[...]
    h1 = jnp.maximum(dis * (s + y1_ref[...]) + b1_ref[...], 0.0)
    z_ref[...] = h1 * dis


def _tc_post(sz_ref, z_ref, dis_ref, w2_ref, b2_ref, batch_ref,
             wf1_ref, bf1_ref, wf2_ref, bf2_ref, wf3_ref, bf3_ref,
             wf4_ref, bf4_ref, out_ref, pooled, cnt):
    i = pl.program_id(0)
    dis = dis_ref[...]
    u = jnp.dot(sz_ref[...] + z_ref[...], w2_ref[...],
                preferred_element_type=jnp.float32)
    h2 = jnp.maximum(dis * u + b2_ref[...], 0.0)
    gid = lax.broadcasted_iota(jnp.int32, (BLK, G), 1)
    oh = (batch_ref[...] == gid).astype(jnp.float32)
    dn = (((0,), (0,)), ((), ()))
    pc = lax.dot_general(oh, h2, dn, preferred_element_type=jnp.float32)
    cc = lax.dot_general(oh, jnp.ones_like(h2), dn,
                         preferred_element_type=jnp.float32)

    @pl.when(i == 0)
    def _():
        pooled[...] = pc
        cnt[...] = cc

    @pl.when(i > 0)
    def _():
        pooled[...] += pc
        cnt[...] += cc

    @pl.when(i == GRID - 1)
    def _():
        m = pooled[...] / jnp.maximum(cnt[...], 1.0)
        a = jnp.maximum(jnp.dot(m, wf1_ref[...],
                                preferred_element_type=jnp.float32)
                        + bf1_ref[...], 0.0)
        a = jnp.maximum(jnp.dot(a, wf2_ref[...],
                                preferred_element_type=jnp.float32)
                        + bf2_ref[...], 0.0)
        a = jnp.maximum(jnp.dot(a, wf3_ref[...],
                                preferred_element_type=jnp.float32)
                        + bf3_ref[...], 0.0)
        out_ref[...] = jnp.dot(a, wf4_ref[...],
                               preferred_element_type=jnp.float32) + bf4_ref[...]


def _full(shape):
    return pl.BlockSpec(shape, lambda i: (0,) * len(shape))


def kernel(x, edge_index, batch, W1, b1, W2, b2,
           Wf1, bf1, Wf2, bf2, Wf3, bf3, Wf4, bf4):
    pad = E_PAD - E
    junk = N + jnp.arange(pad, dtype=jnp.int32) % (N_ACC - N)
    src_p = jnp.concatenate(
        [edge_index[0], jnp.zeros((pad,), jnp.int32)]
    ).reshape(IDX_ROWS, C)
    dst_p = jnp.concatenate(
        [edge_index[1], junk]).reshape(IDX_ROWS, C)

    deg2 = _make_deg()(dst_p)

    y1, dis = pl.pallas_call(
        _tc_pre,
        grid=(GRID,),
        in_specs=[
            pl.BlockSpec((BLK, 128), lambda i: (i, 0)),
            _full((128, 64)),
            pl.BlockSpec((NC, BLK, L), lambda i: (0, i, 0)),
        ],
        out_specs=[
            pl.BlockSpec((BLK, 64), lambda i: (i, 0)),
            pl.BlockSpec((BLK, 1), lambda i: (i, 0)),
        ],
        out_shape=[
            jax.ShapeDtypeStruct((N, 64), jnp.float32),
            jax.ShapeDtypeStruct((N, 1), jnp.float32),
        ],
    )(x, W1, deg2)

    seg64 = _make_seg_sum(64)
    s1 = seg64(y1, src_p, dst_p)

    z = pl.pallas_call(
        _tc_mid,
        grid=(GRID,),
        in_specs=[
            pl.BlockSpec((BLK, 64), lambda i: (i, 0)),
            pl.BlockSpec((BLK, 64), lambda i: (i, 0)),
            pl.BlockSpec((BLK, 1), lambda i: (i, 0)),
            _full((1, 64)),
        ],
        out_specs=pl.BlockSpec((BLK, 64), lambda i: (i, 0)),
        out_shape=jax.ShapeDtypeStruct((N, 64), jnp.float32),
    )(s1, y1, dis, b1.reshape(1, 64))

    sz = seg64(z, src_p, dst_p)

    Wf4p = jnp.pad(Wf4, ((0, 0), (0, 28)))
    bf4p = jnp.pad(bf4, (0, 28)).reshape(1, 128)

    out128 = pl.pallas_call(
        _tc_post,
        grid=(GRID,),
        in_specs=[
            pl.BlockSpec((BLK, 64), lambda i: (i, 0)),
            pl.BlockSpec((BLK, 64), lambda i: (i, 0)),
            pl.BlockSpec((BLK, 1), lambda i: (i, 0)),
            _full((64, 128)),
            _full((1, 128)),
            pl.BlockSpec((BLK, 1), lambda i: (i, 0)),
            _full((128, 256)), _full((1, 256)),
            _full((256, 512)), _full((1, 512)),
            _full((512, 512)), _full((1, 512)),
            _full((512, 128)), _full((1, 128)),
        ],
        out_specs=pl.BlockSpec((G, 128), lambda i: (0, 0)),
        out_shape=jax.ShapeDtypeStruct((G, 128), jnp.float32),
        scratch_shapes=[
            pltpu.VMEM((G, 128), jnp.float32),
            pltpu.VMEM((G, 128), jnp.float32),
        ],
    )(sz, z, dis, W2, b2.reshape(1, 128), batch.reshape(N, 1),
      Wf1, bf1.reshape(1, 256), Wf2, bf2.reshape(1, 512),
      Wf3, bf3.reshape(1, 512), Wf4p, bf4p)

    return out128[:, :100].reshape(G, 10, 10)


# speedup vs baseline: 1.2471x; 1.2471x over previous
import functools

import jax
import jax.numpy as jnp
from jax import lax
from jax.experimental import pallas as pl
from jax.experimental.pallas import tpu as pltpu
from jax.experimental.pallas import tpu_sc as plsc

N = 10000
E = 320000
G = 16
NC, NS, L = 2, 16, 16
NW = NC * NS
C = 128
CPT = 80
CPT0 = 155
CPT1 = 5
N_CHUNKS = NS * (CPT0 + CPT1)
E_PAD = C * N_CHUNKS
IDX_ROWS = N_CHUNKS
ROWS_PT = 640
N_ACC = ROWS_PT * NS
BLK = 1000
GRID = N // BLK


def _sc_mesh():
    return plsc.VectorSubcoreMesh(
        core_axis_name="c", subcore_axis_name="s",
        num_cores=NC, num_subcores=NS)


def _fill(ref, rows, cols, val):
    def row_body(r, _):
        def col_body(j, __):
            ref[r, pl.ds(j * L, L)] = jnp.full((L,), val, jnp.float32)
            return 0
        return lax.fori_loop(0, cols // L, col_body, 0)
    lax.fori_loop(0, rows, row_body, 0)


NB = 5
LA = 3


def _make_seg_sum(D):
    def body(y_hbm, src_hbm, dst_hbm, out_hbm, src_v, dst_v, acc, *rest):
        rows = rest[:NB]
        sem_g = rest[NB:2 * NB]
        sem_s = rest[2 * NB:]
        cid = lax.axis_index("c")
        sid = lax.axis_index("s")
        cpt = jnp.where(cid == 0, CPT0, CPT1)
        base = cid * (NS * CPT0) + sid * cpt
        @pl.when(cid == 0)
        def _():
            pltpu.sync_copy(src_hbm.at[pl.ds(base, CPT0)],
                            src_v.at[pl.ds(0, CPT0)])
            pltpu.sync_copy(dst_hbm.at[pl.ds(base, CPT0)],
                            dst_v.at[pl.ds(0, CPT0)])
        @pl.when(cid == 1)
        def _():
            pltpu.sync_copy(src_hbm.at[pl.ds(base, CPT1)],
                            src_v.at[pl.ds(0, CPT1)])
            pltpu.sync_copy(dst_hbm.at[pl.ds(base, CPT1)],
                            dst_v.at[pl.ds(0, CPT1)])
        _fill(rows[0], C, D, 0.0)
        def zero_stripe(j, _):
            pltpu.sync_copy(rows[0], acc.at[pl.ds(sid * ROWS_PT + j * C, C)])
            return 0
        lax.fori_loop(0, ROWS_PT // C, zero_stripe, 0)
        plsc.subcore_barrier()
        for c0 in range(LA):
            pltpu.async_copy(y_hbm.at[src_v.at[c0]], rows[c0], sem_g[c0])
        def round_body(t, _):
            for b in range(NB):
                ci = t * NB + b
                bg = (b + LA) % NB
                cg = ci + LA
                @pl.when(jnp.logical_and(cg < cpt, cg >= NB))
                def _():
                    pltpu.make_async_copy(
                        rows[bg], acc.at[dst_v.at[cg - NB]], sem_s[bg]).wait()
                @pl.when(cg < cpt)
                def _():
                    pltpu.async_copy(
                        y_hbm.at[src_v.at[cg]], rows[bg], sem_g[bg])
                pltpu.make_async_copy(
                    y_hbm.at[src_v.at[ci]], rows[b], sem_g[b]).wait()
                pltpu.async_copy(
                    rows[b], acc.at[dst_v.at[ci]], sem_s[b], add=True)
            return 0
        lax.fori_loop(0, cpt // NB, round_body, 0)
        for b in range(NB):
            pltpu.make_async_copy(
                rows[b], acc.at[dst_v.at[cpt - NB + b]], sem_s[b]).wait()
        plsc.subcore_barrier()
        def copy_out(j, _):
            r0 = sid * ROWS_PT + j * C
            pltpu.sync_copy(acc.at[pl.ds(r0, C)], rows[0])
            pltpu.sync_copy(rows[0], out_hbm.at[cid, pl.ds(r0, C)])
            return 0
        lax.fori_loop(0, ROWS_PT // C, copy_out, 0)

    return pl.kernel(
        body,
        out_type=jax.ShapeDtypeStruct((NC, N_ACC, D), jnp.float32),
        mesh=_sc_mesh(),
        compiler_params=pltpu.CompilerParams(use_tc_tiling_on_sc=False),
        scratch_types=[
            pltpu.VMEM((CPT0, C), jnp.int32),
            pltpu.VMEM((CPT0, C), jnp.int32),
            pltpu.VMEM_SHARED((N_ACC, D), jnp.float32),
        ] + [pltpu.VMEM((C, D), jnp.float32)] * NB
          + [pltpu.SemaphoreType.DMA] * (2 * NB),
    )


def _make_deg():
    def body(dst_hbm, out_hbm, dst_v, rows_v, acc):
        cid = lax.axis_index("c")
        sid = lax.axis_index("s")
        wid = cid * NS + sid
        pltpu.sync_copy(dst_hbm.at[pl.ds(wid * CPT, CPT)], dst_v)
        _fill(rows_v, C, L, 0.0)
        def zero_stripe(j, _):
            pltpu.sync_copy(rows_v, acc.at[pl.ds(sid * ROWS_PT + j * C, C)])
            return 0
        lax.fori_loop(0, ROWS_PT // C, zero_stripe, 0)
        _fill(rows_v, C, L, 1.0)
        plsc.subcore_barrier()
        def chunk(ci, _):
            pltpu.sync_copy(rows_v, acc.at[dst_v.at[ci]], add=True)
            return 0
        lax.fori_loop(0, CPT, chunk, 0)
        plsc.subcore_barrier()
        def copy_out(j, _):
            r0 = sid * ROWS_PT + j * C
            pltpu.sync_copy(acc.at[pl.ds(r0, C)], rows_v)
            pltpu.sync_copy(rows_v, out_hbm.at[cid, pl.ds(r0, C)])
            return 0
        lax.fori_loop(0, ROWS_PT // C, copy_out, 0)

    return pl.kernel(
        body,
        out_type=jax.ShapeDtypeStruct((NC, N_ACC, L), jnp.float32),
        mesh=_sc_mesh(),
        compiler_params=pltpu.CompilerParams(use_tc_tiling_on_sc=False),
        scratch_types=[
            pltpu.VMEM((CPT, C), jnp.int32),
            pltpu.VMEM((C, L), jnp.float32),
            pltpu.VMEM_SHARED((N_ACC, L), jnp.float32),
        ],
    )


def _tc_pre(x_ref, w1_ref, deg_ref, y1_ref, dis_ref):
    deg = deg_ref[0, :, 0:1] + deg_ref[1, :, 0:1] + 1.0
    dis = lax.rsqrt(deg)
    xw = jnp.dot(x_ref[...], w1_ref[...], preferred_element_type=jnp.float32)
    y1_ref[...] = xw * dis
    dis_ref[...] = dis


def _tc_mid(s1_ref, y1_ref, dis_ref, b1_ref, z_ref):
    s = s1_ref[0] + s1_ref[1]
    dis = dis_ref[...]
    h1 = jnp.maximum(dis * (s + y1_ref[...]) + b1_ref[...], 0.0)
    z_ref[...] = h1 * dis


def _tc_post(sz_ref, z_ref, dis_ref, w2_ref, b2_ref, batch_ref,
             wf1_ref, bf1_ref, wf2_ref, bf2_ref, wf3_ref, bf3_ref,
             wf4_ref, bf4_ref, out_ref, pooled, cnt):
    i = pl.program_id(0)
    dis = dis_ref[...]
    u = jnp.dot(sz_ref[0] + sz_ref[1] + z_ref[...], w2_ref[...],
                preferred_element_type=jnp.float32)
    h2 = jnp.maximum(dis * u + b2_ref[...], 0.0)
    gid = lax.broadcasted_iota(jnp.int32, (BLK, G), 1)
    oh = (batch_ref[...] == gid).astype(jnp.float32)
    dn = (((0,), (0,)), ((), ()))
    pc = lax.dot_general(oh, h2, dn, preferred_element_type=jnp.float32)
    cc = lax.dot_general(oh, jnp.ones_like(h2), dn,
                         preferred_element_type=jnp.float32)

    @pl.when(i == 0)
    def _():
        pooled[...] = pc
        cnt[...] = cc

    @pl.when(i > 0)
    def _():
        pooled[...] += pc
        cnt[...] += cc

    @pl.when(i == GRID - 1)
    def _():
        m = pooled[...] / jnp.maximum(cnt[...], 1.0)
        a = jnp.maximum(jnp.dot(m, wf1_ref[...],
                                preferred_element_type=jnp.float32)
                        + bf1_ref[...], 0.0)
        a = jnp.maximum(jnp.dot(a, wf2_ref[...],
                                preferred_element_type=jnp.float32)
                        + bf2_ref[...], 0.0)
        a = jnp.maximum(jnp.dot(a, wf3_ref[...],
                                preferred_element_type=jnp.float32)
                        + bf3_ref[...], 0.0)
        out_ref[...] = jnp.dot(a, wf4_ref[...],
                               preferred_element_type=jnp.float32) + bf4_ref[...]


def _full(shape):
    return pl.BlockSpec(shape, lambda i: (0,) * len(shape))


def kernel(x, edge_index, batch, W1, b1, W2, b2,
           Wf1, bf1, Wf2, bf2, Wf3, bf3, Wf4, bf4):
    pad = E_PAD - E
    junk = N + jnp.arange(pad, dtype=jnp.int32) % (N_ACC - N)
    src_p = jnp.concatenate(
        [edge_index[0], jnp.zeros((pad,), jnp.int32)]
    ).reshape(IDX_ROWS, C)
    dst_p = jnp.concatenate(
        [edge_index[1], junk]).reshape(IDX_ROWS, C)

    deg2 = _make_deg()(dst_p)

    y1, dis = pl.pallas_call(
        _tc_pre,
        grid=(GRID,),
        in_specs=[
            pl.BlockSpec((BLK, 128), lambda i: (i, 0)),
            _full((128, 64)),
            pl.BlockSpec((NC, BLK, L), lambda i: (0, i, 0)),
        ],
        out_specs=[
            pl.BlockSpec((BLK, 64), lambda i: (i, 0)),
            pl.BlockSpec((BLK, 1), lambda i: (i, 0)),
        ],
        out_shape=[
            jax.ShapeDtypeStruct((N, 64), jnp.float32),
            jax.ShapeDtypeStruct((N, 1), jnp.float32),
        ],
    )(x, W1, deg2)

    seg64 = _make_seg_sum(64)
    s1 = seg64(y1, src_p, dst_p)

    z = pl.pallas_call(
        _tc_mid,
        grid=(GRID,),
        in_specs=[
            pl.BlockSpec((NC, BLK, 64), lambda i: (0, i, 0)),
            pl.BlockSpec((BLK, 64), lambda i: (i, 0)),
            pl.BlockSpec((BLK, 1), lambda i: (i, 0)),
            _full((1, 64)),
        ],
        out_specs=pl.BlockSpec((BLK, 64), lambda i: (i, 0)),
        out_shape=jax.ShapeDtypeStruct((N, 64), jnp.float32),
    )(s1, y1, dis, b1.reshape(1, 64))

    sz = seg64(z, src_p, dst_p)

    Wf4p = jnp.pad(Wf4, ((0, 0), (0, 28)))
    bf4p = jnp.pad(bf4, (0, 28)).reshape(1, 128)

    out128 = pl.pallas_call(
        _tc_post,
        grid=(GRID,),
        in_specs=[
            pl.BlockSpec((NC, BLK, 64), lambda i: (0, i, 0)),
            pl.BlockSpec((BLK, 64), lambda i: (i, 0)),
            pl.BlockSpec((BLK, 1), lambda i: (i, 0)),
            _full((64, 128)),
            _full((1, 128)),
            pl.BlockSpec((BLK, 1), lambda i: (i, 0)),
            _full((128, 256)), _full((1, 256)),
            _full((256, 512)), _full((1, 512)),
            _full((512, 512)), _full((1, 512)),
            _full((512, 128)), _full((1, 128)),
        ],
        out_specs=pl.BlockSpec((G, 128), lambda i: (0, 0)),
        out_shape=jax.ShapeDtypeStruct((G, 128), jnp.float32),
        scratch_shapes=[
            pltpu.VMEM((G, 128), jnp.float32),
            pltpu.VMEM((G, 128), jnp.float32),
        ],
    )(sz, z, dis, W2, b2.reshape(1, 128), batch.reshape(N, 1),
      Wf1, bf1.reshape(1, 256), Wf2, bf2.reshape(1, 512),
      Wf3, bf3.reshape(1, 512), Wf4p, bf4p)

    return out128[:, :100].reshape(G, 10, 10)
